# baseline (speedup 1.0000x reference)
import jax
import jax.numpy as jnp
from jax.experimental import pallas as pl
from jax.experimental.pallas import tpu as pltpu

N = 10000
R = 2
INDIM = 128
OUTDIM = 128

TM = 200
NT = N // TM
NBUF = 3
XC = 2000
NCH = N // XC


def _fused_kernel(x_hbm, w_ref, a_hbm, o_ref, fw_ref, slots, xtmp, sem, xsem):
    m = pl.program_id(0)

    def a_copy(block, slot):
        return pltpu.make_async_copy(
            a_hbm.at[pl.ds(block * TM, TM), :], slots.at[slot], sem.at[slot])

    @pl.when(m == 0)
    def _prologue():
        for i in range(NBUF):
            a_copy(i, i).start()

        def x_copy(c, s):
            return pltpu.make_async_copy(
                x_hbm.at[pl.ds(c * XC, XC), :], xtmp.at[s], xsem.at[s])

        x_copy(0, 0).start()
        for c in range(NCH):
            if c + 1 < NCH:
                x_copy(c + 1, (c + 1) % 2).start()
            x_copy(c, c % 2).wait()
            xc = xtmp[c % 2]
            for r in range(R):
                fw_ref[r * N + c * XC:r * N + (c + 1) * XC, :] = jnp.dot(
                    xc, w_ref[r], preferred_element_type=jnp.float32)

    slot = jax.lax.rem(m, NBUF)
    a_copy(m, slot).wait()
    o_ref[...] = jnp.dot(slots[slot], fw_ref[...],
                         preferred_element_type=jnp.float32)

    @pl.when(m + NBUF < NT)
    def _refill():
        a_copy(m + NBUF, slot).start()


@jax.jit
def kernel(X, A, W_F):
    return pl.pallas_call(
        _fused_kernel,
        grid=(NT,),
        in_specs=[
            pl.BlockSpec(memory_space=pltpu.MemorySpace.HBM),
            pl.BlockSpec((R, INDIM, OUTDIM), lambda m: (0, 0, 0)),
            pl.BlockSpec(memory_space=pltpu.MemorySpace.HBM),
        ],
        out_specs=pl.BlockSpec((TM, OUTDIM), lambda m: (m, 0)),
        out_shape=jax.ShapeDtypeStruct((N, OUTDIM), jnp.float32),
        scratch_shapes=[
            pltpu.VMEM((R * N, OUTDIM), jnp.float32),
            pltpu.VMEM((NBUF, TM, R * N), jnp.float32),
            pltpu.VMEM((2, XC, INDIM), jnp.float32),
            pltpu.SemaphoreType.DMA((NBUF,)),
            pltpu.SemaphoreType.DMA((2,)),
        ],
        compiler_params=pltpu.CompilerParams(
            dimension_semantics=("arbitrary",),
        ),
    )(X, W_F, A)


# speedup vs baseline: 1.0367x; 1.0367x over previous
import jax
import jax.numpy as jnp
from jax.experimental import pallas as pl
from jax.experimental.pallas import tpu as pltpu

N = 10000
R = 2
INDIM = 128
OUTDIM = 128

TM = 200


def _fused_kernel(x_ref, w_ref, a_ref, o_ref, fw_ref):
    @pl.when(pl.program_id(0) == 0)
    def _compute_fw():
        for r in range(R):
            fw_ref[r * N:(r + 1) * N, :] = jnp.dot(
                x_ref[...], w_ref[r],
                preferred_element_type=jnp.float32).astype(jnp.bfloat16)

    o_ref[...] = jnp.dot(a_ref[...].astype(jnp.bfloat16), fw_ref[...],
                         preferred_element_type=jnp.float32)


@jax.jit
def kernel(X, A, W_F):
    return pl.pallas_call(
        _fused_kernel,
        grid=(N // TM,),
        in_specs=[
            pl.BlockSpec((N, INDIM), lambda m: (0, 0)),
            pl.BlockSpec((R, INDIM, OUTDIM), lambda m: (0, 0, 0)),
            pl.BlockSpec((TM, R * N), lambda m: (m, 0)),
        ],
        out_specs=pl.BlockSpec((TM, OUTDIM), lambda m: (m, 0)),
        out_shape=jax.ShapeDtypeStruct((N, OUTDIM), jnp.float32),
        scratch_shapes=[pltpu.VMEM((R * N, OUTDIM), jnp.bfloat16)],
        compiler_params=pltpu.CompilerParams(
            dimension_semantics=("arbitrary",),
        ),
    )(X, W_F, A)


# speedup vs baseline: 1.0381x; 1.0013x over previous
import jax
import jax.numpy as jnp
from jax.experimental import pallas as pl
from jax.experimental.pallas import tpu as pltpu

N = 10000
R = 2
INDIM = 128
OUTDIM = 128

TM = 200


def _fused_kernel(x_ref, w_ref, a_ref, o_ref, fw_ref):
    @pl.when(pl.program_id(0) == 0)
    def _compute_fw():
        for r in range(R):
            fw_ref[r * N:(r + 1) * N, :] = jnp.dot(
                x_ref[...], w_ref[r], preferred_element_type=jnp.float32)

    o_ref[...] = jnp.dot(a_ref[...], fw_ref[...],
                         preferred_element_type=jnp.float32)


@jax.jit
def kernel(X, A, W_F):
    return pl.pallas_call(
        _fused_kernel,
        grid=(N // TM,),
        in_specs=[
            pl.BlockSpec((N, INDIM), lambda m: (0, 0)),
            pl.BlockSpec((R, INDIM, OUTDIM), lambda m: (0, 0, 0)),
            pl.BlockSpec((TM, R * N), lambda m: (m, 0)),
        ],
        out_specs=pl.BlockSpec((TM, OUTDIM), lambda m: (m, 0)),
        out_shape=jax.ShapeDtypeStruct((N, OUTDIM), jnp.float32),
        scratch_shapes=[pltpu.VMEM((R * N, OUTDIM), jnp.float32)],
        compiler_params=pltpu.CompilerParams(
            dimension_semantics=("arbitrary",),
        ),
    )(X, W_F, A)
